# baseline (speedup 1.0000x reference)
import jax
import jax.numpy as jnp
from jax.experimental import pallas as pl
from jax.experimental.pallas import tpu as pltpu

_N = 512
_D = 256
_OUT = 200
_EPS = 1e-5
_NBLK = 4
_BLK = _N // _NBLK

_F32 = jnp.float32


def _dot_t(a, b):
    return jax.lax.dot_general(
        a, b, (((1,), (1,)), ((), ())), preferred_element_type=_F32)


def _dot_tn(a, b):
    return jax.lax.dot_general(
        a, b, (((0,), (0,)), ((), ())), preferred_element_type=_F32)


def _mgcn_body(x_ref, wggl_ref, bggl_ref, w10_ref, b1_ref, w20_ref, w21_ref,
               b2_ref, w30_ref, w31_ref, w32_ref, b3_ref, gamma_ref, beta_ref,
               y1_ref, y2_ref, y3_ref, an_ref,
               attr_scr, row_scr, anorm_scr):
    pid = pl.program_id(0)

    @pl.when(pid == 0)
    def _():
        attr = jax.nn.sigmoid(_dot_t(x_ref[...], wggl_ref[...]) + bggl_ref[...])
        attr_scr[...] = attr
        ones_row = jnp.full((1, _N), 1.0, dtype=_F32)
        row_scr[0:1, :] = jnp.sqrt(_dot_t(ones_row, attr * attr))

    blk = attr_scr[pl.ds(pid * _BLK, _BLK), :]
    attr_all = attr_scr[...]
    nrm_col = jnp.sqrt(jnp.sum(blk * blk, axis=1, keepdims=True))
    g = _dot_t(blk, attr_all)
    adj = g / jnp.maximum(nrm_col * row_scr[0:1, :], 1e-8)
    anb = adj / jnp.max(adj, axis=1, keepdims=True)
    an_ref[...] = anb
    anorm_scr[pl.ds(pid * _BLK, _BLK), :] = anb

    @pl.when(pid == _NBLK - 1)
    def _():
        a_norm = anorm_scr[...]
        xf = x_ref[...]

        ii = jax.lax.broadcasted_iota(jnp.int32, (_N, _N), 0)
        jj = jax.lax.broadcasted_iota(jnp.int32, (_N, _N), 1)
        am = jnp.where(ii == jj, 0.0, a_norm)
        deg = jnp.sum(am, axis=1, keepdims=True)
        dinv = jnp.where(
            deg > 0, jax.lax.rsqrt(jnp.where(deg > 0, deg, 1.0)), 0.0)

        tx1 = -dinv * _dot_tn(am, dinv * xf)
        tx2 = 2.0 * (-dinv * _dot_tn(am, dinv * tx1)) - xf

        h1 = _dot_t(xf, w10_ref[...]) + b1_ref[...]
        h2 = (_dot_t(xf, w20_ref[...]) + _dot_t(tx1, w21_ref[...])
              + b2_ref[...])
        h3 = (_dot_t(xf, w30_ref[...]) + _dot_t(tx1, w31_ref[...])
              + _dot_t(tx2, w32_ref[...]) + b3_ref[...])

        gamma = gamma_ref[...]
        beta = beta_ref[...]

        def _bn(h):
            mu = jnp.mean(h, axis=0, keepdims=True)
            var = jnp.mean((h - mu) * (h - mu), axis=0, keepdims=True)
            return (h - mu) * jax.lax.rsqrt(var + _EPS) * gamma + beta

        y1_ref[...] = _bn(h1)
        y2_ref[...] = _bn(h2)
        y3_ref[...] = _bn(h3)


def kernel(x, W_ggl, b_ggl, W1_0, b1, W2_0, W2_1, b2, W3_0, W3_1, W3_2, b3,
           gamma, beta):
    row = lambda v: v.reshape(1, -1).astype(_F32)
    full = lambda s: pl.BlockSpec(s, lambda i: (0, 0))
    out = pl.pallas_call(
        _mgcn_body,
        grid=(_NBLK,),
        in_specs=[
            full((_N, _D)), full((_N, _D)), full((1, _N)),
            full((_OUT, _D)), full((1, _OUT)),
            full((_OUT, _D)), full((_OUT, _D)), full((1, _OUT)),
            full((_OUT, _D)), full((_OUT, _D)), full((_OUT, _D)),
            full((1, _OUT)),
            full((1, _OUT)), full((1, _OUT)),
        ],
        out_specs=(
            full((_N, _OUT)),
            full((_N, _OUT)),
            full((_N, _OUT)),
            pl.BlockSpec((_BLK, _N), lambda i: (i, 0)),
        ),
        out_shape=(
            jax.ShapeDtypeStruct((_N, _OUT), _F32),
            jax.ShapeDtypeStruct((_N, _OUT), _F32),
            jax.ShapeDtypeStruct((_N, _OUT), _F32),
            jax.ShapeDtypeStruct((_N, _N), _F32),
        ),
        scratch_shapes=[
            pltpu.VMEM((_N, _N), _F32),
            pltpu.VMEM((8, _N), _F32),
            pltpu.VMEM((_N, _N), _F32),
        ],
    )(x, W_ggl, row(b_ggl), W1_0, row(b1), W2_0, W2_1, row(b2),
      W3_0, W3_1, W3_2, row(b3), row(gamma), row(beta))
    return out


# speedup vs baseline: 1.1864x; 1.1864x over previous
import jax
import jax.numpy as jnp
from jax.experimental import pallas as pl

_N = 512
_D = 256
_OUT = 200
_EPS = 1e-5

_F32 = jnp.float32


def _dot_t(a, b):
    return jax.lax.dot_general(
        a, b, (((1,), (1,)), ((), ())), preferred_element_type=_F32)


def _dot_tn(a, b):
    return jax.lax.dot_general(
        a, b, (((0,), (0,)), ((), ())), preferred_element_type=_F32)


def _mgcn_body(x_ref, wggl_ref, bggl_ref, w10_ref, b1_ref, w20_ref, w21_ref,
               b2_ref, w30_ref, w31_ref, w32_ref, b3_ref, gamma_ref, beta_ref,
               y1_ref, y2_ref, y3_ref, an_ref):
    xf = x_ref[...]

    attr = jax.nn.sigmoid(_dot_t(xf, wggl_ref[...]) + bggl_ref[...])

    sq = attr * attr
    sq_col = jnp.sum(sq, axis=1, keepdims=True)
    ones_row = jnp.full((1, _N), 1.0, dtype=_F32)
    sq_row = _dot_t(ones_row, sq)
    nrm_col = jnp.sqrt(sq_col)
    nrm_row = jnp.sqrt(sq_row)
    gram = _dot_t(attr, attr)
    adj = gram / jnp.maximum(nrm_col * nrm_row, 1e-8)

    a_norm = adj / jnp.max(adj, axis=1, keepdims=True)
    an_ref[...] = a_norm

    ii = jax.lax.broadcasted_iota(jnp.int32, (_N, _N), 0)
    jj = jax.lax.broadcasted_iota(jnp.int32, (_N, _N), 1)
    am = jnp.where(ii == jj, 0.0, a_norm)
    deg = jnp.sum(am, axis=1, keepdims=True)
    dinv = jnp.where(deg > 0, jax.lax.rsqrt(jnp.where(deg > 0, deg, 1.0)), 0.0)

    tx1 = -dinv * _dot_tn(am, dinv * xf)
    tx2 = 2.0 * (-dinv * _dot_tn(am, dinv * tx1)) - xf

    h1 = _dot_t(xf, w10_ref[...]) + b1_ref[...]
    h2 = _dot_t(xf, w20_ref[...]) + _dot_t(tx1, w21_ref[...]) + b2_ref[...]
    h3 = (_dot_t(xf, w30_ref[...]) + _dot_t(tx1, w31_ref[...])
          + _dot_t(tx2, w32_ref[...]) + b3_ref[...])

    gamma = gamma_ref[...]
    beta = beta_ref[...]

    def _bn(h):
        mu = jnp.mean(h, axis=0, keepdims=True)
        var = jnp.mean((h - mu) * (h - mu), axis=0, keepdims=True)
        return (h - mu) * jax.lax.rsqrt(var + _EPS) * gamma + beta

    y1_ref[...] = _bn(h1)
    y2_ref[...] = _bn(h2)
    y3_ref[...] = _bn(h3)


def kernel(x, W_ggl, b_ggl, W1_0, b1, W2_0, W2_1, b2, W3_0, W3_1, W3_2, b3,
           gamma, beta):
    row = lambda v: v.reshape(1, -1).astype(_F32)
    out = pl.pallas_call(
        _mgcn_body,
        out_shape=(
            jax.ShapeDtypeStruct((_N, _OUT), _F32),
            jax.ShapeDtypeStruct((_N, _OUT), _F32),
            jax.ShapeDtypeStruct((_N, _OUT), _F32),
            jax.ShapeDtypeStruct((_N, _N), _F32),
        ),
    )(x, W_ggl, row(b_ggl), W1_0, row(b1), W2_0, W2_1, row(b2),
      W3_0, W3_1, W3_2, row(b3), row(gamma), row(beta))
    return out
